# baseline (speedup 1.0000x reference)
import functools

import numpy as np
import jax
import jax.numpy as jnp
from jax import lax
from jax.experimental import pallas as pl
from jax.experimental.pallas import tpu as pltpu
from jax.experimental.pallas import tpu_sc as plsc

_NC = 2
_NS = 16
_NW = _NC * _NS
_LANES = 16
_CH = 100
_NBUF = 8


def _pe_table(seq_len, d):
    pos = jnp.arange(seq_len, dtype=jnp.float32)[:, None]
    div = jnp.exp(jnp.arange(0, d, 2, dtype=jnp.float32)
                  * (-np.log(10000.0) / d))
    pe = jnp.zeros((seq_len, d), dtype=jnp.float32)
    pe = pe.at[:, 0::2].set(jnp.sin(pos * div))
    pe = pe.at[:, 1::2].set(jnp.cos(pos * div))
    return pe


@functools.partial(jax.jit, static_argnames=())
def kernel(X, emb_table):
    B, L = X.shape
    V, D = emb_table.shape
    BL = B * L
    assert BL % (_NW * _CH) == 0 and _CH % L == 0
    per_w = BL // _NW
    nchunk = per_w // _CH
    ngroup = nchunk // _NBUF
    assert nchunk % _NBUF == 0

    idx2 = X.reshape(BL // _CH, _CH)
    pe2 = jnp.concatenate([_pe_table(L, D)] * (_CH // L), axis=0)

    mesh = plsc.VectorSubcoreMesh(core_axis_name="c", subcore_axis_name="s")

    def body(table, idx, pe_in, out, idx_v, pe_v, bufs, gsems, osems):
        wid = lax.axis_index("s") * _NC + lax.axis_index("c")
        c0 = wid * nchunk
        pltpu.sync_copy(idx.at[pl.ds(c0, nchunk)], idx_v)
        pltpu.sync_copy(pe_in, pe_v)
        for g in range(ngroup):
            for b in range(_NBUF):
                c = g * _NBUF + b
                pltpu.async_copy(table.at[idx_v.at[c]], bufs.at[b], gsems[b])
            for b in range(_NBUF):
                c = g * _NBUF + b
                pltpu.make_async_copy(
                    table.at[idx_v.at[c]], bufs.at[b], gsems[b]).wait()

                def add_pe(j, _, b=b):
                    for k in range(D // _LANES):
                        sl = pl.ds(k * _LANES, _LANES)
                        plsc.addupdate(bufs.at[b, j, sl], pe_v[j, sl])
                    return 0
                lax.fori_loop(0, _CH, add_pe, 0, unroll=4)
                row0 = (c0 + c) * _CH
                pltpu.async_copy(
                    bufs.at[b], out.at[pl.ds(row0, _CH)], osems[b])
            for b in range(_NBUF):
                c = g * _NBUF + b
                row0 = (c0 + c) * _CH
                pltpu.make_async_copy(
                    bufs.at[b], out.at[pl.ds(row0, _CH)], osems[b]).wait()

    run = pl.kernel(
        body,
        out_type=jax.ShapeDtypeStruct((BL, D), jnp.float32),
        mesh=mesh,
        compiler_params=pltpu.CompilerParams(use_tc_tiling_on_sc=False),
        scratch_types=[
            pltpu.VMEM((nchunk, _CH), jnp.int32),
            pltpu.VMEM((_CH, D), jnp.float32),
            pltpu.VMEM((_NBUF, _CH, D), jnp.float32),
            [pltpu.SemaphoreType.DMA] * _NBUF,
            [pltpu.SemaphoreType.DMA] * _NBUF,
        ],
    )
    out = run(emb_table, idx2, pe2)
    return out.reshape(B, L, D)


# speedup vs baseline: 1.0332x; 1.0332x over previous
import numpy as np
import jax
import jax.numpy as jnp
from jax import lax
from jax.experimental import pallas as pl
from jax.experimental.pallas import tpu as pltpu
from jax.experimental.pallas import tpu_sc as plsc

_NC = 2
_NS = 16
_NW = _NC * _NS
_LANES = 16
_CH = 100
_NBUF = 4


def _pe_table(seq_len, d):
    pos = jnp.arange(seq_len, dtype=jnp.float32)[:, None]
    div = jnp.exp(jnp.arange(0, d, 2, dtype=jnp.float32)
                  * (-np.log(10000.0) / d))
    pe = jnp.zeros((seq_len, d), dtype=jnp.float32)
    pe = pe.at[:, 0::2].set(jnp.sin(pos * div))
    pe = pe.at[:, 1::2].set(jnp.cos(pos * div))
    return pe


@jax.jit
def kernel(X, emb_table):
    B, L = X.shape
    V, D = emb_table.shape
    BL = B * L
    spc = _CH // L
    nchunk = BL // (_NW * _CH)
    assert BL % (_NW * _CH) == 0 and _CH % L == 0 and nchunk % _NBUF == 0

    idx3 = X.reshape(_NW, nchunk, _CH)
    pe2 = jnp.concatenate([_pe_table(L, D)] * spc, axis=0)
    tabp = jnp.pad(emb_table, ((0, 0), (0, 128 - D)))

    mesh = plsc.VectorSubcoreMesh(core_axis_name="c", subcore_axis_name="s")

    def body(table, idx, pe_in, out, idx_v, pe_v, raw, fin, gsems, osems):
        wid = lax.axis_index("s") * _NC + lax.axis_index("c")
        pltpu.sync_copy(idx.at[wid], idx_v)
        pltpu.sync_copy(pe_in, pe_v)
        for g in range(nchunk // _NBUF):
            for b in range(_NBUF):
                c = g * _NBUF + b
                pltpu.async_copy(
                    table.at[idx_v.at[c, pl.ds(0, _CH)]], raw.at[b],
                    gsems[b])
            for b in range(_NBUF):
                c = g * _NBUF + b
                pltpu.make_async_copy(
                    table.at[idx_v.at[c, pl.ds(0, _CH)]], raw.at[b],
                    gsems[b]).wait()

                def add_pe(j, _, b=b):
                    for k in range(D // _LANES):
                        sl = pl.ds(k * _LANES, _LANES)
                        fin[b, j, sl] = raw[b, j, sl] + pe_v[j, sl]
                    return 0
                lax.fori_loop(0, _CH, add_pe, 0, unroll=4)

                seq0 = (wid * nchunk + c) * spc
                for h in range(spc):
                    pltpu.async_copy(
                        fin.at[b, pl.ds(h * L, L)], out.at[seq0 + h],
                        osems[b])
            for b in range(_NBUF):
                c = g * _NBUF + b
                seq0 = (wid * nchunk + c) * spc
                for h in range(spc):
                    pltpu.make_async_copy(
                        fin.at[b, pl.ds(h * L, L)], out.at[seq0 + h],
                        osems[b]).wait()

    run = pl.kernel(
        body,
        out_type=jax.ShapeDtypeStruct((B, L, D), jnp.float32),
        mesh=mesh,
        compiler_params=pltpu.CompilerParams(use_tc_tiling_on_sc=True),
        scratch_types=[
            pltpu.VMEM((nchunk, _CH), jnp.int32),
            pltpu.VMEM((_CH, D), jnp.float32),
            pltpu.VMEM((_NBUF, _CH, 128), jnp.float32),
            pltpu.VMEM((_NBUF, _CH, D), jnp.float32),
            [pltpu.SemaphoreType.DMA] * _NBUF,
            [pltpu.SemaphoreType.DMA] * _NBUF,
        ],
    )
    return run(tabp, idx3, pe2)


# speedup vs baseline: 1.1053x; 1.0698x over previous
import numpy as np
import jax
import jax.numpy as jnp
from jax import lax
from jax.experimental import pallas as pl
from jax.experimental.pallas import tpu as pltpu
from jax.experimental.pallas import tpu_sc as plsc

_NC = 2
_NS = 16
_NW = _NC * _NS
_LANES = 16
_CH = 100
_NBUF = 4


def _pe_table(seq_len, d):
    pos = jnp.arange(seq_len, dtype=jnp.float32)[:, None]
    div = jnp.exp(jnp.arange(0, d, 2, dtype=jnp.float32)
                  * (-np.log(10000.0) / d))
    pe = jnp.zeros((seq_len, d), dtype=jnp.float32)
    pe = pe.at[:, 0::2].set(jnp.sin(pos * div))
    pe = pe.at[:, 1::2].set(jnp.cos(pos * div))
    return pe


@jax.jit
def kernel(X, emb_table):
    B, L = X.shape
    V, D = emb_table.shape
    BL = B * L
    spc = _CH // L
    nchunk = BL // (_NW * _CH)
    assert BL % (_NW * _CH) == 0 and _CH % L == 0 and nchunk % _NBUF == 0

    idx3 = X.reshape(_NW, nchunk, _CH)
    pe2 = jnp.concatenate([_pe_table(L, D)] * spc, axis=0)
    tabp = jnp.pad(emb_table, ((0, 0), (0, 128 - D)))

    mesh = plsc.VectorSubcoreMesh(core_axis_name="c", subcore_axis_name="s")

    def body(table, idx, pe_in, out, idx_v, pe_v, raw, fin, gsems, osems):
        wid = lax.axis_index("s") * _NC + lax.axis_index("c")
        pltpu.sync_copy(idx.at[wid], idx_v)
        pltpu.sync_copy(pe_in, pe_v)

        def gather(c):
            b = c % _NBUF
            return pltpu.async_copy(
                table.at[idx_v.at[c, pl.ds(0, _CH)]], raw.at[b], gsems[b])

        def write(c, h):
            b = c % _NBUF
            seq0 = (wid * nchunk + c) * spc
            return pltpu.async_copy(
                fin.at[b, pl.ds(h * L, L)], out.at[seq0 + h], osems[b])

        for c in range(_NBUF - 1):
            gather(c)
        for c in range(nchunk):
            b = c % _NBUF
            nb = c + _NBUF - 1
            if nb < nchunk:
                gather(nb)
            pltpu.make_async_copy(
                table.at[idx_v.at[c, pl.ds(0, _CH)]], raw.at[b],
                gsems[b]).wait()
            if c >= _NBUF:
                for h in range(spc):
                    seq0 = (wid * nchunk + (c - _NBUF)) * spc
                    pltpu.make_async_copy(
                        fin.at[b, pl.ds(h * L, L)], out.at[seq0 + h],
                        osems[b]).wait()

            def add_pe(j, _, b=b):
                for k in range(D // _LANES):
                    sl = pl.ds(k * _LANES, _LANES)
                    fin[b, j, sl] = raw[b, j, sl] + pe_v[j, sl]
                return 0
            lax.fori_loop(0, _CH, add_pe, 0, unroll=4)
            for h in range(spc):
                write(c, h)
        for c in range(nchunk - _NBUF, nchunk):
            b = c % _NBUF
            seq0 = (wid * nchunk + c) * spc
            for h in range(spc):
                pltpu.make_async_copy(
                    fin.at[b, pl.ds(h * L, L)], out.at[seq0 + h],
                    osems[b]).wait()

    run = pl.kernel(
        body,
        out_type=jax.ShapeDtypeStruct((B, L, D), jnp.float32),
        mesh=mesh,
        compiler_params=pltpu.CompilerParams(use_tc_tiling_on_sc=True),
        scratch_types=[
            pltpu.VMEM((nchunk, _CH), jnp.int32),
            pltpu.VMEM((_CH, D), jnp.float32),
            pltpu.VMEM((_NBUF, _CH, 128), jnp.float32),
            pltpu.VMEM((_NBUF, _CH, D), jnp.float32),
            [pltpu.SemaphoreType.DMA] * _NBUF,
            [pltpu.SemaphoreType.DMA] * _NBUF,
        ],
    )
    return run(tabp, idx3, pe2)


# speedup vs baseline: 1.1059x; 1.0005x over previous
import functools

import numpy as np
import jax
import jax.numpy as jnp
from jax import lax
from jax.experimental import pallas as pl
from jax.experimental.layout import Format, Layout
from jax.experimental.pallas import tpu as pltpu
from jax.experimental.pallas import tpu_sc as plsc

_NC = 2
_NS = 16
_NW = _NC * _NS
_LANES = 16
_CH = 100
_NBUF = 4


def _pe_table(seq_len, d):
    pos = jnp.arange(seq_len, dtype=jnp.float32)[:, None]
    div = jnp.exp(jnp.arange(0, d, 2, dtype=jnp.float32)
                  * (-np.log(10000.0) / d))
    pe = jnp.zeros((seq_len, d), dtype=jnp.float32)
    pe = pe.at[:, 0::2].set(jnp.sin(pos * div))
    pe = pe.at[:, 1::2].set(jnp.cos(pos * div))
    return pe


_OUT_FORMAT = Format(Layout(major_to_minor=(0, 1, 2)),
                     jax.sharding.SingleDeviceSharding(jax.devices()[0]))


@functools.partial(jax.jit, out_shardings=_OUT_FORMAT)
def kernel(X, emb_table):
    B, L = X.shape
    V, D = emb_table.shape
    BL = B * L
    spc = _CH // L
    nchunk = BL // (_NW * _CH)
    assert BL % (_NW * _CH) == 0 and _CH % L == 0 and nchunk % _NBUF == 0

    idx3 = X.reshape(_NW, nchunk, _CH)
    pe2 = jnp.concatenate([_pe_table(L, D)] * spc, axis=0)
    tabp = jnp.pad(emb_table, ((0, 0), (0, 128 - D)))

    mesh = plsc.VectorSubcoreMesh(core_axis_name="c", subcore_axis_name="s")

    def body(table, idx, pe_in, out, idx_v, pe_v, raw, fin, gsems, osems):
        wid = lax.axis_index("s") * _NC + lax.axis_index("c")
        pltpu.sync_copy(idx.at[wid], idx_v)
        pltpu.sync_copy(pe_in, pe_v)

        def gather(c):
            b = c % _NBUF
            return pltpu.async_copy(
                table.at[idx_v.at[c, pl.ds(0, _CH)]], raw.at[b], gsems[b])

        def write(c, h):
            b = c % _NBUF
            seq0 = (wid * nchunk + c) * spc
            return pltpu.async_copy(
                fin.at[b, pl.ds(h * L, L)], out.at[seq0 + h], osems[b])

        for c in range(_NBUF - 1):
            gather(c)
        for c in range(nchunk):
            b = c % _NBUF
            nb = c + _NBUF - 1
            if nb < nchunk:
                gather(nb)
            pltpu.make_async_copy(
                table.at[idx_v.at[c, pl.ds(0, _CH)]], raw.at[b],
                gsems[b]).wait()
            if c >= _NBUF:
                for h in range(spc):
                    seq0 = (wid * nchunk + (c - _NBUF)) * spc
                    pltpu.make_async_copy(
                        fin.at[b, pl.ds(h * L, L)], out.at[seq0 + h],
                        osems[b]).wait()

            def add_pe(j, _, b=b):
                for k in range(D // _LANES):
                    sl = pl.ds(k * _LANES, _LANES)
                    fin[b, j, sl] = raw[b, j, sl] + pe_v[j, sl]
                return 0
            lax.fori_loop(0, _CH, add_pe, 0, unroll=4)
            for h in range(spc):
                write(c, h)
        for c in range(nchunk - _NBUF, nchunk):
            b = c % _NBUF
            seq0 = (wid * nchunk + c) * spc
            for h in range(spc):
                pltpu.make_async_copy(
                    fin.at[b, pl.ds(h * L, L)], out.at[seq0 + h],
                    osems[b]).wait()

    run = pl.kernel(
        body,
        out_type=jax.ShapeDtypeStruct((B, L, D), jnp.float32),
        mesh=mesh,
        compiler_params=pltpu.CompilerParams(use_tc_tiling_on_sc=True),
        scratch_types=[
            pltpu.VMEM((nchunk, _CH), jnp.int32),
            pltpu.VMEM((_CH, D), jnp.float32),
            pltpu.VMEM((_NBUF, _CH, 128), jnp.float32),
            pltpu.VMEM((_NBUF, _CH, D), jnp.float32),
            [pltpu.SemaphoreType.DMA] * _NBUF,
            [pltpu.SemaphoreType.DMA] * _NBUF,
        ],
    )
    return run(tabp, idx3, pe2)
